# Initial kernel scaffold; baseline (speedup 1.0000x reference)
import jax
import jax.numpy as jnp
from jax.experimental import pallas as pl


def kernel(x, edge_index, edge_attr, W_msg, W_node, W_mol_a, W_mol_b, W_ffn, b_ffn):
    raise NotImplementedError("write your pallas kernel here")


# speedup vs baseline: 1.4817x; 1.4817x over previous
import jax
import jax.numpy as jnp
from jax import lax
from jax.experimental import pallas as pl
from jax.experimental.pallas import tpu as pltpu
from jax.experimental.pallas import tpu_sc as plsc

N = 10000
E = 320000
D = 128
DE = 16

NC = 2
NS = 16
NW = NC * NS
EW = E // NW
C = 80
NCHUNK = EW // C
NPAD = 10240
RPT = NPAD // NS
ZD = RPT // C

RB = 2000
NBB = E // RB
NAB = N // RB


NH = 5120
NHP = 5248
TRASH = 5200
ZROWS = NHP // NS
DROWS = NH // NS


def _remap(dst_v, idxp_v, lo):
    for u in range(C // 16):
        su = pl.ds(u * 16, 16)
        t = dst_v[su] - lo
        ok = (t >= 0) & (t < NH)
        idxp_v[su] = jnp.where(ok, t, TRASH)


def _zero_acc(zeros_hbm, agg_sp, sid):
    r0 = sid * ZROWS
    for d in range(ZROWS // C):
        pltpu.sync_copy(zeros_hbm, agg_sp.at[pl.ds(r0 + d * C, C)])
    rem = ZROWS % C
    if rem:
        pltpu.sync_copy(zeros_hbm.at[pl.ds(0, rem)],
                        agg_sp.at[pl.ds(r0 + (ZROWS // C) * C, rem)])


def _dump_acc(agg_sp, agg_hbm, cid, sid, out_base):
    for d in range(DROWS // C):
        r0 = sid * DROWS + d * C
        pltpu.sync_copy(agg_sp.at[pl.ds(r0, C)],
                        agg_hbm.at[cid, pl.ds(out_base + r0, C)])


def _edge_sc(y_hbm, z_hbm, src_hbm, dst_hbm, zeros_hbm, msg_hbm, agg_hbm,
             src_v, dst_v, idxp_v, yg_v, z_v, m_v, agg_sp, sem):
    cid = lax.axis_index("c")
    sid = lax.axis_index("s")
    wid = sid * NC + cid
    _zero_acc(zeros_hbm, agg_sp, sid)
    plsc.subcore_barrier()
    base0 = wid * EW

    @pl.loop(0, NCHUNK)
    def _chunk(k):
        base = base0 + k * C
        pltpu.sync_copy(src_hbm.at[pl.ds(base, C)], src_v)
        pltpu.sync_copy(dst_hbm.at[pl.ds(base, C)], dst_v)
        pltpu.async_copy(y_hbm.at[src_v], yg_v, sem).wait()
        pltpu.sync_copy(z_hbm.at[pl.ds(base, C)], z_v)

        @pl.loop(0, C)
        def _row(r):
            for j in range(D // 16):
                s = pl.ds(j * 16, 16)
                m_v[r, s] = jnp.maximum(yg_v[r, s] + z_v[r, s], 0.0)

        pltpu.sync_copy(m_v, msg_hbm.at[pl.ds(base, C)])
        _remap(dst_v, idxp_v, 0)
        pltpu.sync_copy(m_v, agg_sp.at[idxp_v], add=True)

    plsc.subcore_barrier()
    _dump_acc(agg_sp, agg_hbm, cid, sid, 0)
    plsc.subcore_barrier()
    _zero_acc(zeros_hbm, agg_sp, sid)
    plsc.subcore_barrier()

    @pl.loop(0, NCHUNK)
    def _chunk2(k):
        base = base0 + k * C
        pltpu.sync_copy(dst_hbm.at[pl.ds(base, C)], dst_v)
        pltpu.sync_copy(msg_hbm.at[pl.ds(base, C)], m_v)
        _remap(dst_v, idxp_v, NH)
        pltpu.sync_copy(m_v, agg_sp.at[idxp_v], add=True)

    plsc.subcore_barrier()
    _dump_acc(agg_sp, agg_hbm, cid, sid, NH)


def _edge_phase(y, z, src_r, dst_r, zeros):
    mesh = plsc.VectorSubcoreMesh(core_axis_name="c", subcore_axis_name="s")
    return pl.kernel(
        _edge_sc,
        out_type=(
            jax.ShapeDtypeStruct((E, D), jnp.float32),
            jax.ShapeDtypeStruct((NC, NPAD, D), jnp.float32),
        ),
        mesh=mesh,
        scratch_types=[
            pltpu.VMEM((C,), jnp.int32),
            pltpu.VMEM((C,), jnp.int32),
            pltpu.VMEM((C,), jnp.int32),
            pltpu.VMEM((C, D), jnp.float32),
            pltpu.VMEM((C, D), jnp.float32),
            pltpu.VMEM((C, D), jnp.float32),
            pltpu.VMEM_SHARED((NHP, D), jnp.float32),
            pltpu.SemaphoreType.DMA,
        ],
    )(y, z, src_r, dst_r, zeros)


def _prep_y(x, W1):
    def body(x_ref, w_ref, y_ref):
        y_ref[...] = jnp.dot(x_ref[...], w_ref[...],
                             preferred_element_type=jnp.float32)

    return pl.pallas_call(
        body,
        out_shape=jax.ShapeDtypeStruct((N, D), jnp.float32),
    )(x, W1)


def _prep_z(ea, W2):
    def body(ea_ref, w_ref, z_ref):
        z_ref[...] = jnp.dot(ea_ref[...], w_ref[...],
                             preferred_element_type=jnp.float32)

    return pl.pallas_call(
        body,
        grid=(E // RB,),
        in_specs=[
            pl.BlockSpec((RB, DE), lambda i: (i, 0)),
            pl.BlockSpec((DE, D), lambda i: (0, 0)),
        ],
        out_specs=pl.BlockSpec((RB, D), lambda i: (i, 0)),
        out_shape=jax.ShapeDtypeStruct((E, D), jnp.float32),
    )(ea, W2)


def _atoms_phase(agg2, x, W_node, W_mol_a, W_mol_b, W_ffn, b_ffn_2d):
    def body(agg_ref, x_ref, wn_ref, wa_ref, wb_ref, wf_ref,
             bf_ref, atoms_ref, mv_ref, c_ref):
        agg = agg_ref[0, :N, :] + agg_ref[1, :N, :]
        pre = jnp.dot(agg, wn_ref[...],
                      preferred_element_type=jnp.float32) + x_ref[...]
        atoms = jnp.maximum(pre, 0.0)
        atoms_ref[...] = atoms
        mean_a = jnp.sum(atoms, axis=0, keepdims=True) * (1.0 / N)
        mean_b = jnp.sum(agg, axis=0, keepdims=True) * (1.0 / E)
        mva = jnp.dot(mean_a, wa_ref[...], preferred_element_type=jnp.float32)
        mvb = jnp.dot(mean_b, wb_ref[...], preferred_element_type=jnp.float32)
        mv_ref[0:1, :] = mvb
        mv_ref[1:2, :] = mva
        w2 = wf_ref[D:, :]
        c_ref[0:1, :] = jnp.dot(mvb, w2,
                                preferred_element_type=jnp.float32) + bf_ref[...]
        c_ref[1:2, :] = jnp.dot(mva, w2,
                                preferred_element_type=jnp.float32) + bf_ref[...]

    return pl.pallas_call(
        body,
        out_shape=(
            jax.ShapeDtypeStruct((N, D), jnp.float32),
            jax.ShapeDtypeStruct((2, D), jnp.float32),
            jax.ShapeDtypeStruct((2, 1), jnp.float32),
        ),
    )(agg2, x, W_node, W_mol_a, W_mol_b, W_ffn, b_ffn_2d)


def _fill_phase(msg, atoms_v, mv, c, w1):
    def body(msg_ref, at_ref, mv_ref, c_ref, w1_ref,
             v_ref, r_ref, t_ref):
        i = pl.program_id(0)
        is_bond = i < NBB
        blk = jnp.where(is_bond, msg_ref[...], at_ref[...])
        mvrow = jnp.where(is_bond, mv_ref[0:1, :], mv_ref[1:2, :])
        cc = jnp.where(is_bond, c_ref[0:1, :], c_ref[1:2, :])
        v_ref[:, :D] = blk
        v_ref[:, D:] = jnp.broadcast_to(mvrow, (RB, D))
        t_ref[...] = jnp.sum(blk, axis=1, keepdims=True)
        r_ref[...] = jnp.dot(blk, w1_ref[...],
                             preferred_element_type=jnp.float32) + cc

    return pl.pallas_call(
        body,
        grid=(NBB + NAB,),
        in_specs=[
            pl.BlockSpec((RB, D), lambda i: (jnp.minimum(i, NBB - 1), 0)),
            pl.BlockSpec((RB, D), lambda i: (jnp.maximum(i - NBB, 0), 0)),
            pl.BlockSpec((2, D), lambda i: (0, 0)),
            pl.BlockSpec((2, 1), lambda i: (0, 0)),
            pl.BlockSpec((D, 1), lambda i: (0, 0)),
        ],
        out_specs=[
            pl.BlockSpec((RB, 2 * D), lambda i: (i, 0)),
            pl.BlockSpec((RB, 1), lambda i: (i, 0)),
            pl.BlockSpec((RB, 1), lambda i: (i, 0)),
        ],
        out_shape=[
            jax.ShapeDtypeStruct((E + N, 2 * D), jnp.float32),
            jax.ShapeDtypeStruct((E + N, 1), jnp.float32),
            jax.ShapeDtypeStruct((E + N, 1), jnp.float32),
        ],
    )(msg, atoms_v, mv, c, w1)


def kernel(x, edge_index, edge_attr, W_msg, W_node, W_mol_a, W_mol_b, W_ffn,
           b_ffn):
    x = x.astype(jnp.float32)
    ei = edge_index.astype(jnp.int32)
    src_r = ei[0]
    dst_r = ei[1]
    W1 = W_msg[:D]
    W2 = W_msg[D:]
    y = _prep_y(x, W1)
    z = _prep_z(edge_attr, W2)
    zeros = jnp.zeros((C, D), jnp.float32)
    msg, agg2 = _edge_phase(y, z, src_r, dst_r, zeros)
    atoms_v, mv, c = _atoms_phase(agg2, x, W_node, W_mol_a, W_mol_b,
                                  W_ffn, jnp.reshape(b_ffn, (1, 1)))
    w1 = W_ffn[:D]
    v_all, r_all, t_cat = _fill_phase(msg, atoms_v, mv, c, w1)
    t_all = jnp.concatenate([t_cat[:E], t_cat[E + 1:]], axis=0)
    return (r_all, t_all, v_all)
